# baseline (speedup 1.0000x reference)
import functools

import jax
import jax.numpy as jnp
from jax import lax
from jax.experimental import pallas as pl
from jax.experimental.pallas import tpu as pltpu
from jax.experimental.pallas import tpu_sc as plsc

B = 16384
F = 26
V = 100000
D = 32
NUM = 13

NC = 2
NS = 16
NW = NC * NS
ROWS = F * D
QB = B // 4


def _gather_body(tab_ref, xcat_ref, feats_ref, idx_v, row_v, out0, out1,
                 sem0, sem1):
    wid = lax.axis_index("s") * NC + lax.axis_index("c")
    is2 = wid < 12
    f = jnp.where(is2, wid // 2, wid - 6)
    d0 = jnp.where(is2, (wid % 2) * 16, 0)
    nd = jnp.where(is2, 16, 32)

    pltpu.sync_copy(xcat_ref.at[f], idx_v)

    outs = (out0, out1)
    sems = (sem0, sem1)

    def do_row(r, _):
        d = d0 + r
        pltpu.sync_copy(tab_ref.at[f, d, :], row_v)

        for q in range(4):
            ob = outs[q % 2]
            sm = sems[q % 2]
            base = q * QB

            if q >= 2:
                pltpu.make_async_copy(
                    tab_ref.at[f, d, pl.ds(0, QB // 2)], ob, sm).wait()
            else:
                @pl.when(r > 0)
                def _drain():
                    pltpu.make_async_copy(
                        tab_ref.at[f, d, pl.ds(0, QB // 2)], ob, sm).wait()

            @plsc.parallel_loop(0, QB, 32, unroll=8)
            def _gather(i):
                v0 = plsc.load_gather(row_v, [idx_v[pl.ds(base + i, 16)]])
                v1 = plsc.load_gather(row_v, [idx_v[pl.ds(base + i + 16, 16)]])
                packed = plsc.pack(v0, v1, format=plsc.PackFormat.INTERLEAVED)
                ob[pl.ds(i // 2, 16)] = plsc.bitcast(packed, jnp.int32)

            pltpu.async_copy(
                ob, feats_ref.at[f * D + d, pl.ds(base // 2, QB // 2)], sm)

        return _

    lax.fori_loop(0, nd, do_row, 0, unroll=False)

    pltpu.make_async_copy(tab_ref.at[f, d0, pl.ds(0, QB // 2)], out0, sem0).wait()
    pltpu.make_async_copy(tab_ref.at[f, d0, pl.ds(0, QB // 2)], out1, sem1).wait()


_gather = functools.partial(
    pl.kernel,
    out_type=jax.ShapeDtypeStruct((ROWS, B // 2), jnp.int32),
    mesh=plsc.VectorSubcoreMesh(core_axis_name="c", subcore_axis_name="s"),
    compiler_params=pltpu.CompilerParams(needs_layout_passes=False),
    scratch_types=[
        pltpu.VMEM((B,), jnp.int32),
        pltpu.VMEM((V,), jnp.float32),
        pltpu.VMEM((QB // 2,), jnp.int32),
        pltpu.VMEM((QB // 2,), jnp.int32),
        pltpu.SemaphoreType.DMA,
        pltpu.SemaphoreType.DMA,
    ],
)(_gather_body)


def _mlp_body(f_ref, xne_ref, xno_ref, w1c_ref, w1n_ref, b1_ref, w2_ref,
              b2_ref, w3_ref, b3_ref, o_ref):
    raw = f_ref[...]
    fe = jax.lax.bitcast_convert_type(raw << 16, jnp.float32)
    fo = jax.lax.bitcast_convert_type(raw & jnp.int32(-65536), jnp.float32)

    def head(feats, xn):
        h = jnp.dot(w1c_ref[...], feats, preferred_element_type=jnp.float32)
        h += jnp.dot(w1n_ref[...], xn, preferred_element_type=jnp.float32)
        h = jnp.maximum(h + b1_ref[...], 0.0)
        h = jnp.maximum(
            jnp.dot(w2_ref[...], h, preferred_element_type=jnp.float32)
            + b2_ref[...], 0.0)
        return (jnp.dot(w3_ref[...], h, preferred_element_type=jnp.float32)
                + b3_ref[...])

    o_ref[0:1, :] = head(fe, xne_ref[...])
    o_ref[1:2, :] = head(fo, xno_ref[...])


BT = 2048


def _mlp(featsT, x_numT_e, x_numT_o, w1cT, w1nT, b1, w2T, b2, w3T, b3):
    grid = (B // BT,)
    return pl.pallas_call(
        _mlp_body,
        grid=grid,
        in_specs=[
            pl.BlockSpec((ROWS, BT // 2), lambda i: (0, i)),
            pl.BlockSpec((NUM, BT // 2), lambda i: (0, i)),
            pl.BlockSpec((NUM, BT // 2), lambda i: (0, i)),
            pl.BlockSpec((128, ROWS), lambda i: (0, 0)),
            pl.BlockSpec((128, NUM), lambda i: (0, 0)),
            pl.BlockSpec((128, 1), lambda i: (0, 0)),
            pl.BlockSpec((64, 128), lambda i: (0, 0)),
            pl.BlockSpec((64, 1), lambda i: (0, 0)),
            pl.BlockSpec((1, 64), lambda i: (0, 0)),
            pl.BlockSpec((1, 1), lambda i: (0, 0)),
        ],
        out_specs=pl.BlockSpec((2, BT // 2), lambda i: (0, i)),
        out_shape=jax.ShapeDtypeStruct((2, B // 2), jnp.float32),
    )(featsT, x_numT_e, x_numT_o, w1cT, w1nT, b1, w2T, b2, w3T, b3)


def kernel(x_cat, x_num, tables, W1, b1, W2, b2, W3, b3):
    tabT = tables.transpose(0, 2, 1)
    xcatT = (x_cat.T.reshape(F, B // 32, 16, 2)
             .transpose(0, 1, 3, 2).reshape(F, B))
    featsT = _gather(tabT, xcatT)
    xnT = x_num.T
    out2 = _mlp(featsT, xnT[:, 0::2], xnT[:, 1::2], W1[: F * D].T,
                W1[F * D:].T,
                b1.reshape(128, 1), W2.T, b2.reshape(64, 1), W3.T,
                b3.reshape(1, 1))
    return out2.T.reshape(B)


# speedup vs baseline: 1.0723x; 1.0723x over previous
import functools

import jax
import jax.numpy as jnp
from jax import lax
from jax.experimental import pallas as pl
from jax.experimental.pallas import tpu as pltpu
from jax.experimental.pallas import tpu_sc as plsc

B = 16384
F = 26
V = 100000
D = 32
NUM = 13

NC = 2
NS = 16
NW = NC * NS
ROWS = F * D
QB = B // 4


def _gather_body(tab_ref, xcat_ref, feats_ref, idx_v, row_v, out0, out1,
                 sem0, sem1):
    wid = lax.axis_index("s") * NC + lax.axis_index("c")
    is2 = wid < 12
    f = jnp.where(is2, wid // 2, wid - 6)
    d0 = jnp.where(is2, (wid % 2) * 16, 0)
    nd = jnp.where(is2, 16, 32)

    pltpu.sync_copy(xcat_ref.at[f], idx_v)

    outs = (out0, out1)
    sems = (sem0, sem1)

    def do_row(r, _):
        d = d0 + r
        pltpu.sync_copy(tab_ref.at[f, d, :], row_v)

        for q in range(4):
            ob = outs[q % 2]
            sm = sems[q % 2]
            base = q * QB

            if q >= 2:
                pltpu.make_async_copy(
                    tab_ref.at[f, d, pl.ds(0, QB // 2)], ob, sm).wait()
            else:
                @pl.when(r > 0)
                def _drain():
                    pltpu.make_async_copy(
                        tab_ref.at[f, d, pl.ds(0, QB // 2)], ob, sm).wait()

            @plsc.parallel_loop(0, QB, 32, unroll=8)
            def _gather(i):
                v0 = plsc.load_gather(row_v, [idx_v[pl.ds(base + i, 16)]])
                v1 = plsc.load_gather(row_v, [idx_v[pl.ds(base + i + 16, 16)]])
                packed = plsc.pack(v0, v1, format=plsc.PackFormat.INTERLEAVED)
                ob[pl.ds(i // 2, 16)] = plsc.bitcast(packed, jnp.int32)

            pltpu.async_copy(
                ob, feats_ref.at[f * D + d, pl.ds(base // 2, QB // 2)], sm)

        return _

    lax.fori_loop(0, nd, do_row, 0, unroll=False)

    pltpu.make_async_copy(tab_ref.at[f, d0, pl.ds(0, QB // 2)], out0, sem0).wait()
    pltpu.make_async_copy(tab_ref.at[f, d0, pl.ds(0, QB // 2)], out1, sem1).wait()


_gather = functools.partial(
    pl.kernel,
    out_type=jax.ShapeDtypeStruct((ROWS, B // 2), jnp.int32),
    mesh=plsc.VectorSubcoreMesh(core_axis_name="c", subcore_axis_name="s"),
    compiler_params=pltpu.CompilerParams(needs_layout_passes=False),
    scratch_types=[
        pltpu.VMEM((B,), jnp.int32),
        pltpu.VMEM((V,), jnp.float32),
        pltpu.VMEM((QB // 2,), jnp.int32),
        pltpu.VMEM((QB // 2,), jnp.int32),
        pltpu.SemaphoreType.DMA,
        pltpu.SemaphoreType.DMA,
    ],
)(_gather_body)


def _mlp_body(f_ref, xne_ref, xno_ref, w1c_ref, w1n_ref, b1_ref, w2_ref,
              b2_ref, w3_ref, b3_ref, o_ref):
    raw = f_ref[...]
    fe = jax.lax.bitcast_convert_type(raw << 16, jnp.float32)
    fo = jax.lax.bitcast_convert_type(raw & jnp.int32(-65536), jnp.float32)

    def head(feats, xn):
        h = jnp.dot(w1c_ref[...], feats, preferred_element_type=jnp.float32)
        h += jnp.dot(w1n_ref[...], xn, preferred_element_type=jnp.float32)
        h = jnp.maximum(h + b1_ref[...], 0.0)
        h = jnp.maximum(
            jnp.dot(w2_ref[...], h, preferred_element_type=jnp.float32)
            + b2_ref[...], 0.0)
        return (jnp.dot(w3_ref[...], h, preferred_element_type=jnp.float32)
                + b3_ref[...])

    o_ref[0:1, :] = head(fe, xne_ref[...])
    o_ref[1:2, :] = head(fo, xno_ref[...])


BT = 2048


def _mlp(featsT, x_numT_e, x_numT_o, w1cT, w1nT, b1, w2T, b2, w3T, b3):
    grid = (B // BT,)
    return pl.pallas_call(
        _mlp_body,
        grid=grid,
        in_specs=[
            pl.BlockSpec((ROWS, BT // 2), lambda i: (0, i)),
            pl.BlockSpec((NUM, BT // 2), lambda i: (0, i)),
            pl.BlockSpec((NUM, BT // 2), lambda i: (0, i)),
            pl.BlockSpec((128, ROWS), lambda i: (0, 0)),
            pl.BlockSpec((128, NUM), lambda i: (0, 0)),
            pl.BlockSpec((128, 1), lambda i: (0, 0)),
            pl.BlockSpec((64, 128), lambda i: (0, 0)),
            pl.BlockSpec((64, 1), lambda i: (0, 0)),
            pl.BlockSpec((1, 64), lambda i: (0, 0)),
            pl.BlockSpec((1, 1), lambda i: (0, 0)),
        ],
        out_specs=pl.BlockSpec((2, BT // 2), lambda i: (0, i)),
        out_shape=jax.ShapeDtypeStruct((2, B // 2), jnp.float32),
    )(featsT, x_numT_e, x_numT_o, w1cT, w1nT, b1, w2T, b2, w3T, b3)


def kernel(x_cat, x_num, tables, W1, b1, W2, b2, W3, b3):
    tabT = tables.transpose(0, 2, 1)
    xcatT = x_cat.T
    featsT = _gather(tabT, xcatT)
    xn3 = x_num.T.reshape(NUM, B // 32, 2, 16)
    out2 = _mlp(featsT, xn3[:, :, 0, :].reshape(NUM, B // 2),
                xn3[:, :, 1, :].reshape(NUM, B // 2), W1[: F * D].T,
                W1[F * D:].T,
                b1.reshape(128, 1), W2.T, b2.reshape(64, 1), W3.T,
                b3.reshape(1, 1))
    return out2.reshape(2, B // 32, 16).transpose(1, 0, 2).reshape(B)


# speedup vs baseline: 1.0757x; 1.0031x over previous
import functools

import jax
import jax.numpy as jnp
from jax import lax
from jax.experimental import pallas as pl
from jax.experimental.pallas import tpu as pltpu
from jax.experimental.pallas import tpu_sc as plsc

B = 16384
F = 26
V = 100000
D = 32
NUM = 13

NC = 2
NS = 16
NW = NC * NS
ROWS = F * D
QB = B // 4


def _gather_body(tab_ref, xcat_ref, feats_ref, idx_v, row_v, out0, out1,
                 sem0, sem1):
    wid = lax.axis_index("s") * NC + lax.axis_index("c")
    is2 = wid < 12
    f = jnp.where(is2, wid // 2, wid - 6)
    d0 = jnp.where(is2, (wid % 2) * 16, 0)
    nd = jnp.where(is2, 16, 32)

    pltpu.sync_copy(xcat_ref.at[f], idx_v)

    outs = (out0, out1)
    sems = (sem0, sem1)

    def do_row(r, _):
        d = d0 + r
        pltpu.sync_copy(tab_ref.at[f, d, :], row_v)

        for q in range(4):
            ob = outs[q % 2]
            sm = sems[q % 2]
            base = q * QB

            if q >= 2:
                pltpu.make_async_copy(
                    tab_ref.at[f, d, pl.ds(0, QB)], ob, sm).wait()
            else:
                @pl.when(r > 0)
                def _drain():
                    pltpu.make_async_copy(
                        tab_ref.at[f, d, pl.ds(0, QB)], ob, sm).wait()

            @plsc.parallel_loop(0, QB, 16, unroll=8)
            def _gather(i):
                ob[pl.ds(i, 16)] = plsc.load_gather(
                    row_v, [idx_v[pl.ds(base + i, 16)]])

            pltpu.async_copy(ob, feats_ref.at[f * D + d, pl.ds(base, QB)], sm)

        return _

    lax.fori_loop(0, nd, do_row, 0, unroll=False)

    pltpu.make_async_copy(tab_ref.at[f, d0, pl.ds(0, QB)], out0, sem0).wait()
    pltpu.make_async_copy(tab_ref.at[f, d0, pl.ds(0, QB)], out1, sem1).wait()


_gather = functools.partial(
    pl.kernel,
    out_type=jax.ShapeDtypeStruct((ROWS, B), jnp.float32),
    mesh=plsc.VectorSubcoreMesh(core_axis_name="c", subcore_axis_name="s"),
    compiler_params=pltpu.CompilerParams(needs_layout_passes=False),
    scratch_types=[
        pltpu.VMEM((B,), jnp.int32),
        pltpu.VMEM((V,), jnp.float32),
        pltpu.VMEM((QB,), jnp.float32),
        pltpu.VMEM((QB,), jnp.float32),
        pltpu.SemaphoreType.DMA,
        pltpu.SemaphoreType.DMA,
    ],
)(_gather_body)


def _mlp_body(f_ref, xn_ref, w1c_ref, w1n_ref, b1_ref, w2_ref, b2_ref,
              w3_ref, b3_ref, o_ref):
    h = jnp.dot(w1c_ref[...], f_ref[...], preferred_element_type=jnp.float32)
    h += jnp.dot(w1n_ref[...], xn_ref[...], preferred_element_type=jnp.float32)
    h = jnp.maximum(h + b1_ref[...], 0.0)
    h = jnp.maximum(
        jnp.dot(w2_ref[...], h, preferred_element_type=jnp.float32) + b2_ref[...],
        0.0,
    )
    o_ref[...] = (
        jnp.dot(w3_ref[...], h, preferred_element_type=jnp.float32) + b3_ref[...]
    )


BT = 4096


def _mlp(featsT, x_numT, w1cT, w1nT, b1, w2T, b2, w3T, b3):
    grid = (B // BT,)
    return pl.pallas_call(
        _mlp_body,
        grid=grid,
        in_specs=[
            pl.BlockSpec((ROWS, BT), lambda i: (0, i)),
            pl.BlockSpec((NUM, BT), lambda i: (0, i)),
            pl.BlockSpec((128, ROWS), lambda i: (0, 0)),
            pl.BlockSpec((128, NUM), lambda i: (0, 0)),
            pl.BlockSpec((128, 1), lambda i: (0, 0)),
            pl.BlockSpec((64, 128), lambda i: (0, 0)),
            pl.BlockSpec((64, 1), lambda i: (0, 0)),
            pl.BlockSpec((1, 64), lambda i: (0, 0)),
            pl.BlockSpec((1, 1), lambda i: (0, 0)),
        ],
        out_specs=pl.BlockSpec((1, BT), lambda i: (0, i)),
        out_shape=jax.ShapeDtypeStruct((1, B), jnp.float32),
    )(featsT, x_numT, w1cT, w1nT, b1, w2T, b2, w3T, b3)


def kernel(x_cat, x_num, tables, W1, b1, W2, b2, W3, b3):
    tabT = tables.transpose(0, 2, 1)
    xcatT = x_cat.T
    featsT = _gather(tabT, xcatT)
    out = _mlp(featsT, x_num.T, W1[: F * D].T, W1[F * D:].T,
               b1.reshape(128, 1), W2.T, b2.reshape(64, 1), W3.T,
               b3.reshape(1, 1))
    return out[0]


# speedup vs baseline: 1.0812x; 1.0051x over previous
import functools

import jax
import jax.numpy as jnp
from jax import lax
from jax.experimental import pallas as pl
from jax.experimental.pallas import tpu as pltpu
from jax.experimental.pallas import tpu_sc as plsc

B = 16384
F = 26
V = 100000
D = 32
NUM = 13

NC = 2
NS = 16
NW = NC * NS
ROWS = F * D
QB = B // 4


def _gather_body(tab_ref, xcat_ref, feats_ref, idx_v, row_v, out0, out1,
                 sem0, sem1):
    wid = lax.axis_index("s") * NC + lax.axis_index("c")
    is2 = wid < 12
    f = jnp.where(is2, wid // 2, wid - 6)
    d0 = jnp.where(is2, (wid % 2) * 16, 0)
    nd = jnp.where(is2, 16, 32)

    pltpu.sync_copy(xcat_ref.at[f], idx_v)

    outs = (out0, out1)
    sems = (sem0, sem1)

    def do_row(r, _):
        d = d0 + r
        pltpu.sync_copy(tab_ref.at[f, d, :], row_v)

        for q in range(4):
            ob = outs[q % 2]
            sm = sems[q % 2]
            base = q * QB

            if q >= 2:
                pltpu.make_async_copy(
                    tab_ref.at[f, d, pl.ds(0, QB)], ob, sm).wait()
            else:
                @pl.when(r > 0)
                def _drain():
                    pltpu.make_async_copy(
                        tab_ref.at[f, d, pl.ds(0, QB)], ob, sm).wait()

            @plsc.parallel_loop(0, QB, 16, unroll=8)
            def _gather(i):
                ob[pl.ds(i, 16)] = plsc.load_gather(
                    row_v, [idx_v[pl.ds(base + i, 16)]])

            pltpu.async_copy(ob, feats_ref.at[f * D + d, pl.ds(base, QB)], sm)

        return _

    lax.fori_loop(0, nd, do_row, 0, unroll=False)

    pltpu.make_async_copy(tab_ref.at[f, d0, pl.ds(0, QB)], out0, sem0).wait()
    pltpu.make_async_copy(tab_ref.at[f, d0, pl.ds(0, QB)], out1, sem1).wait()


_gather = functools.partial(
    pl.kernel,
    out_type=jax.ShapeDtypeStruct((ROWS, B), jnp.float32),
    mesh=plsc.VectorSubcoreMesh(core_axis_name="c", subcore_axis_name="s"),
    compiler_params=pltpu.CompilerParams(needs_layout_passes=False),
    scratch_types=[
        pltpu.VMEM((B,), jnp.int32),
        pltpu.VMEM((V,), jnp.float32),
        pltpu.VMEM((QB,), jnp.float32),
        pltpu.VMEM((QB,), jnp.float32),
        pltpu.SemaphoreType.DMA,
        pltpu.SemaphoreType.DMA,
    ],
)(_gather_body)


def _mlp_body(f_ref, xn_ref, w1c_ref, w1n_ref, b1_ref, w2_ref, b2_ref,
              w3_ref, b3_ref, o_ref):
    h = jnp.dot(w1c_ref[...], f_ref[...], preferred_element_type=jnp.float32)
    h += jnp.dot(w1n_ref[...], xn_ref[...], preferred_element_type=jnp.float32)
    h = jnp.maximum(h + b1_ref[...], 0.0)
    h = jnp.maximum(
        jnp.dot(w2_ref[...], h, preferred_element_type=jnp.float32) + b2_ref[...],
        0.0,
    )
    o_ref[...] = (
        jnp.dot(w3_ref[...], h, preferred_element_type=jnp.float32) + b3_ref[...]
    )


BT = 2048


def _mlp(featsT, x_numT, w1cT, w1nT, b1, w2T, b2, w3T, b3):
    grid = (B // BT,)
    return pl.pallas_call(
        _mlp_body,
        grid=grid,
        in_specs=[
            pl.BlockSpec((ROWS, BT), lambda i: (0, i)),
            pl.BlockSpec((NUM, BT), lambda i: (0, i)),
            pl.BlockSpec((128, ROWS), lambda i: (0, 0)),
            pl.BlockSpec((128, NUM), lambda i: (0, 0)),
            pl.BlockSpec((128, 1), lambda i: (0, 0)),
            pl.BlockSpec((64, 128), lambda i: (0, 0)),
            pl.BlockSpec((64, 1), lambda i: (0, 0)),
            pl.BlockSpec((1, 64), lambda i: (0, 0)),
            pl.BlockSpec((1, 1), lambda i: (0, 0)),
        ],
        out_specs=pl.BlockSpec((1, BT), lambda i: (0, i)),
        out_shape=jax.ShapeDtypeStruct((1, B), jnp.float32),
    )(featsT, x_numT, w1cT, w1nT, b1, w2T, b2, w3T, b3)


def kernel(x_cat, x_num, tables, W1, b1, W2, b2, W3, b3):
    tabT = tables.transpose(0, 2, 1)
    xcatT = x_cat.T
    featsT = _gather(tabT, xcatT)
    out = _mlp(featsT, x_num.T, W1[: F * D].T, W1[F * D:].T,
               b1.reshape(128, 1), W2.T, b2.reshape(64, 1), W3.T,
               b3.reshape(1, 1))
    return out[0]
